# Initial kernel scaffold; baseline (speedup 1.0000x reference)
import jax
import jax.numpy as jnp
from jax.experimental import pallas as pl


def kernel(adj, x, pseudo, W0, b0, W1, root1, bias1, W2, root2, bias2, W3, root3, bias3, W4, b4, W5, b5):
    raise NotImplementedError("write your pallas kernel here")


# speedup vs baseline: 3.3515x; 3.3515x over previous
import functools

import jax
import jax.numpy as jnp
from jax import lax
from jax.experimental import pallas as pl
from jax.experimental.pallas import tpu as pltpu
from jax.experimental.pallas import tpu_sc as plsc

N = 10000
E = 160000
K = 5
KK = K * K
NCORES = 2
NSUB = 16
NW = NCORES * NSUB
EBLK = 64
EP = 161792
EPW = EP // NW

_F32 = jnp.float32
_I32 = jnp.int32


def _prep_basis(src, px, py):
    def body(src_ref, px_ref, py_ref, idx_ref, w_ref):
        s = src_ref[:]
        p0 = px_ref[:] * (K - 1.0)
        p1 = py_ref[:] * (K - 1.0)
        lo0 = jnp.clip(jnp.floor(p0), 0.0, K - 2.0)
        lo1 = jnp.clip(jnp.floor(p1), 0.0, K - 2.0)
        f0 = p0 - lo0
        f1 = p1 - lo1
        kb = lo0.astype(_I32) * K + lo1.astype(_I32)
        idx_ref[0, :] = kb * N + s
        idx_ref[1, :] = (kb + 1) * N + s
        idx_ref[2, :] = (kb + K) * N + s
        idx_ref[3, :] = (kb + K + 1) * N + s
        g0 = 1.0 - f0
        g1 = 1.0 - f1
        w_ref[0, :] = g0 * g1
        w_ref[1, :] = g0 * f1
        w_ref[2, :] = f0 * g1
        w_ref[3, :] = f0 * f1

    return pl.pallas_call(
        body,
        out_shape=[
            jax.ShapeDtypeStruct((4, E), _I32),
            jax.ShapeDtypeStruct((4, E), _F32),
        ],
    )(src, px, py)


def _mm(a, w, b, act=None, bn=400):
    n, kin = a.shape
    m = w.shape[1]
    if b is None:
        b = jnp.zeros((m,), _F32)

    def body(a_ref, w_ref, b_ref, o_ref):
        acc = jnp.dot(a_ref[:], w_ref[:], preferred_element_type=_F32,
                      precision=lax.Precision.HIGHEST)
        acc = acc + b_ref[:]
        if act == "elu":
            acc = jnp.where(acc > 0.0, acc, (jnp.exp(acc) - 1.0))
        o_ref[:] = acc

    return pl.pallas_call(
        body,
        grid=(n // bn,),
        in_specs=[
            pl.BlockSpec((bn, kin), lambda i: (i, 0)),
            pl.BlockSpec((kin, m), lambda i: (0, 0)),
            pl.BlockSpec((1, m), lambda i: (0, 0)),
        ],
        out_specs=pl.BlockSpec((bn, m), lambda i: (i, 0)),
        out_shape=jax.ShapeDtypeStruct((n, m), _F32),
    )(a, w, b.reshape(1, m))


def _mm_table(h, wpad, wacc, one_col=None, bn=400):
    n, fin = h.shape

    def body(a_ref, w_ref, o_ref):
        acc = jnp.dot(a_ref[:], w_ref[0], preferred_element_type=_F32,
                      precision=lax.Precision.HIGHEST)
        if one_col is not None:
            lane = lax.broadcasted_iota(_I32, (bn, wacc), 1)
            acc = acc + (lane == one_col).astype(_F32)
        o_ref[0] = acc

    return pl.pallas_call(
        body,
        grid=(n // bn, KK),
        in_specs=[
            pl.BlockSpec((bn, fin), lambda i, k: (i, 0)),
            pl.BlockSpec((1, fin, wacc), lambda i, k: (k, 0, 0)),
        ],
        out_specs=pl.BlockSpec((1, bn, wacc), lambda i, k: (k, i, 0)),
        out_shape=jax.ShapeDtypeStruct((KK, n, wacc), _F32),
    )(h, wpad)


def _post(p0, p1, h, root, bias, fout, rdeg=None):
    bn = 400
    first = rdeg is None

    def body(p0_ref, p1_ref, h_ref, root_ref, b_ref, *refs):
        agg = p0_ref[:] + p1_ref[:]
        if first:
            o_ref, rdeg_ref = refs
            rd = 1.0 / jnp.maximum(agg[:, fout:fout + 1], 1.0)
            rdeg_ref[:] = rd
        else:
            rd_ref, o_ref = refs[0], refs[1]
            rd = rd_ref[:]
        acc = agg[:, :fout] * rd
        acc = acc + jnp.dot(h_ref[:], root_ref[:], preferred_element_type=_F32,
                            precision=lax.Precision.HIGHEST)
        acc = acc + b_ref[:]
        o_ref[:] = jnp.where(acc > 0.0, acc, (jnp.exp(acc) - 1.0))

    wacc = p0.shape[1]
    fin = h.shape[1]
    in_specs = [
        pl.BlockSpec((bn, wacc), lambda i: (i, 0)),
        pl.BlockSpec((bn, wacc), lambda i: (i, 0)),
        pl.BlockSpec((bn, fin), lambda i: (i, 0)),
        pl.BlockSpec((fin, fout), lambda i: (0, 0)),
        pl.BlockSpec((1, fout), lambda i: (0, 0)),
    ]
    args = [p0, p1, h, root, bias.reshape(1, fout)]
    if first:
        out_specs = [
            pl.BlockSpec((bn, fout), lambda i: (i, 0)),
            pl.BlockSpec((bn, 1), lambda i: (i, 0)),
        ]
        out_shape = [
            jax.ShapeDtypeStruct((N, fout), _F32),
            jax.ShapeDtypeStruct((N, 1), _F32),
        ]
    else:
        in_specs.append(pl.BlockSpec((bn, 1), lambda i: (i, 0)))
        args.append(rdeg)
        out_specs = pl.BlockSpec((bn, fout), lambda i: (i, 0))
        out_shape = jax.ShapeDtypeStruct((N, fout), _F32)

    return pl.pallas_call(
        body,
        grid=(N // bn,),
        in_specs=in_specs,
        out_specs=out_specs,
        out_shape=out_shape,
    )(*args)


def _final_logsoftmax(a, w, b):
    n, kin = a.shape
    m = w.shape[1]
    bn = 400

    def body(a_ref, w_ref, b_ref, o_ref):
        logits = jnp.dot(a_ref[:], w_ref[:], preferred_element_type=_F32,
                         precision=lax.Precision.HIGHEST)
        logits = logits + b_ref[:]
        mx = jnp.max(logits, axis=1, keepdims=True)
        sh = logits - mx
        lse = jnp.log(jnp.sum(jnp.exp(sh), axis=1, keepdims=True))
        o_ref[:] = sh - lse

    return pl.pallas_call(
        body,
        grid=(n // bn,),
        in_specs=[
            pl.BlockSpec((bn, kin), lambda i: (i, 0)),
            pl.BlockSpec((kin, m), lambda i: (0, 0)),
            pl.BlockSpec((1, m), lambda i: (0, 0)),
        ],
        out_specs=pl.BlockSpec((bn, m), lambda i: (i, 0)),
        out_shape=jax.ShapeDtypeStruct((n, m), _F32),
    )(a, w, b.reshape(1, m))


def _sc_scatter(y, g0, g1, g2, g3, w0, w1, w2, w3, dstc, wacc, npass):
    mesh = plsc.VectorSubcoreMesh(core_axis_name="c", subcore_axis_name="s")
    nvec = wacc // 16
    nh = ((N + npass - 1) // npass + 15) // 16 * 16
    rpsh = nh // NSUB
    nacc = nh + 8

    zfull = rpsh // 16 * 16
    zrem = rpsh - zfull

    def body(y_hbm, g0_h, g1_h, g2_h, g3_h, w0_h, w1_h, w2_h, w3_h, dst_h,
             out_hbm, spill_hbm, i0, i1, i2, i3, wv0, wv1, wv2, wv3, dstv,
             r0, r1, r2, r3, msgv, zb, acc, sem):
        cid = lax.axis_index("c")
        sid = lax.axis_index("s")
        wid = sid * NCORES + cid
        base = wid * EPW

        @pl.loop(0, 16)
        def _(r):
            for c in range(nvec):
                zb[r, pl.ds(c * 16, 16)] = jnp.zeros((16,), _F32)

        for p in range(npass):
            zoff = sid * rpsh

            @pl.loop(0, zfull, step=16)
            def _(r):
                pltpu.sync_copy(zb, acc.at[pl.ds(zoff + r, 16)])

            if zrem:
                pltpu.sync_copy(zb.at[pl.ds(0, zrem)],
                                acc.at[pl.ds(zoff + zfull, zrem)])
            plsc.subcore_barrier()

            @pl.loop(0, EPW, step=EBLK)
            def _(j):
                s = base + j
                pltpu.sync_copy(dst_h.at[pl.ds(s, EBLK)], dstv)
                if p == 0:
                    pltpu.sync_copy(g0_h.at[pl.ds(s, EBLK)], i0)
                    pltpu.sync_copy(g1_h.at[pl.ds(s, EBLK)], i1)
                    pltpu.sync_copy(g2_h.at[pl.ds(s, EBLK)], i2)
                    pltpu.sync_copy(g3_h.at[pl.ds(s, EBLK)], i3)
                    pltpu.sync_copy(w0_h.at[pl.ds(s, EBLK)], wv0)
                    pltpu.sync_copy(w1_h.at[pl.ds(s, EBLK)], wv1)
                    pltpu.sync_copy(w2_h.at[pl.ds(s, EBLK)], wv2)
                    pltpu.sync_copy(w3_h.at[pl.ds(s, EBLK)], wv3)
                    d0 = pltpu.async_copy(y_hbm.at[i0], r0, sem)
                    d1 = pltpu.async_copy(y_hbm.at[i1], r1, sem)
                    d2 = pltpu.async_copy(y_hbm.at[i2], r2, sem)
                    d3 = pltpu.async_copy(y_hbm.at[i3], r3, sem)
                    d0.wait()
                    d1.wait()
                    d2.wait()
                    d3.wait()

                    @pl.loop(0, EBLK, step=16)
                    def _(e0):
                        vw0 = wv0[pl.ds(e0, 16)]
                        vw1 = wv1[pl.ds(e0, 16)]
                        vw2 = wv2[pl.ds(e0, 16)]
                        vw3 = wv3[pl.ds(e0, 16)]
                        for jj in range(16):
                            e = e0 + jj
                            a0 = vw0[jj]
                            a1 = vw1[jj]
                            a2 = vw2[jj]
                            a3 = vw3[jj]
                            for c in range(nvec):
                                sl = pl.ds(c * 16, 16)
                                msgv[e, sl] = (a0 * r0[e, sl] + a1 * r1[e, sl]
                                               + a2 * r2[e, sl]
                                               + a3 * r3[e, sl])

                    pltpu.sync_copy(msgv, spill_hbm.at[pl.ds(s, EBLK)])
                else:
                    pltpu.sync_copy(spill_hbm.at[pl.ds(s, EBLK)], msgv)

                @pl.loop(0, EBLK, step=16)
                def _(e0):
                    d = dstv[pl.ds(e0, 16)] - p * nh
                    inr = (d >= 0) & (d < nh)
                    dstv[pl.ds(e0, 16)] = jnp.where(inr, d, nh)

                pltpu.sync_copy(msgv, acc.at[dstv], add=True)

            plsc.subcore_barrier()
            pltpu.sync_copy(acc.at[pl.ds(sid * rpsh, rpsh)],
                            out_hbm.at[cid, p, sid])
            plsc.subcore_barrier()

    kfn = pl.kernel(
        body,
        out_type=[
            jax.ShapeDtypeStruct((NCORES, npass, NSUB, rpsh, wacc), _F32),
            jax.ShapeDtypeStruct((EP, wacc), _F32),
        ],
        mesh=mesh,
        scratch_types=[
            pltpu.VMEM((EBLK,), _I32),
            pltpu.VMEM((EBLK,), _I32),
            pltpu.VMEM((EBLK,), _I32),
            pltpu.VMEM((EBLK,), _I32),
            pltpu.VMEM((EBLK,), _F32),
            pltpu.VMEM((EBLK,), _F32),
            pltpu.VMEM((EBLK,), _F32),
            pltpu.VMEM((EBLK,), _F32),
            pltpu.VMEM((EBLK,), _I32),
            pltpu.VMEM((EBLK, 128), _F32),
            pltpu.VMEM((EBLK, 128), _F32),
            pltpu.VMEM((EBLK, 128), _F32),
            pltpu.VMEM((EBLK, 128), _F32),
            pltpu.VMEM((EBLK, wacc), _F32),
            pltpu.VMEM((16, wacc), _F32),
            pltpu.VMEM_SHARED((nacc, wacc), _F32),
            pltpu.SemaphoreType.DMA,
        ],
    )
    return kfn(y, g0, g1, g2, g3, w0, w1, w2, w3, dstc)


def _spline_layer(h, gparts, wparts, dstc, W, root, bias, rdeg, npass):
    fout = W.shape[2]
    first = rdeg is None
    wacc = fout + 16 if first else fout
    wpad = jnp.pad(W, ((0, 0), (0, 0), (0, 128 - fout)))
    one_col = fout if first else None
    y = _mm_table(h, wpad, 128, one_col=one_col).reshape(KK * N, 128)
    parts, _ = _sc_scatter(y, *gparts, *wparts, dstc, wacc, npass)
    nrows = parts.shape[1] * parts.shape[2] * parts.shape[3]
    parts = parts.reshape(NCORES, nrows, wacc)[:, :N, :]
    return _post(parts[0], parts[1], h, root, bias, fout, rdeg=rdeg)


def kernel(adj, x, pseudo, W0, b0, W1, root1, bias1, W2, root2, bias2,
           W3, root3, bias3, W4, b4, W5, b5):
    src = adj[0].astype(_I32)
    dstc = adj[1].astype(_I32)
    px = pseudo[:, 0].astype(_F32)
    py = pseudo[:, 1].astype(_F32)

    gidx, gw = _prep_basis(src, px, py)
    pad = EP - E
    spread = (jnp.arange(pad, dtype=_I32) * 97) % (KK * N)
    gparts = tuple(jnp.concatenate([gidx[t], spread]) for t in range(4))
    wparts = tuple(jnp.pad(gw[t], (0, pad)) for t in range(4))
    dstp = jnp.pad(dstc, (0, pad), constant_values=N)

    h = _mm(x, W0, b0, act="elu")
    h, rdeg = _spline_layer(h, gparts, wparts, dstp, W1, root1, bias1, None, 3)
    h = _spline_layer(h, gparts, wparts, dstp, W2, root2, bias2, rdeg, 2)
    h = _spline_layer(h, gparts, wparts, dstp, W3, root3, bias3, rdeg, 2)
    h = _mm(h, W4, b4, act="elu")

    mpad = 6912
    w5p = jnp.pad(W5, ((0, 0), (0, mpad - 6890)))
    b5p = jnp.concatenate([b5, jnp.full((mpad - 6890,), -1e30, _F32)])
    out = _final_logsoftmax(h, w5p, b5p)
    return out[:, :6890]


# speedup vs baseline: 3.4565x; 1.0313x over previous
import functools

import jax
import jax.numpy as jnp
from jax import lax
from jax.experimental import pallas as pl
from jax.experimental.pallas import tpu as pltpu
from jax.experimental.pallas import tpu_sc as plsc

N = 10000
E = 160000
K = 5
KK = K * K
NCORES = 2
NSUB = 16
NW = NCORES * NSUB
EBLK = 64
EP = 161792
EPW = EP // NW

_F32 = jnp.float32
_I32 = jnp.int32


def _prep_basis(src, px, py):
    def body(src_ref, px_ref, py_ref, idx_ref, w_ref):
        s = src_ref[:]
        p0 = px_ref[:] * (K - 1.0)
        p1 = py_ref[:] * (K - 1.0)
        lo0 = jnp.clip(jnp.floor(p0), 0.0, K - 2.0)
        lo1 = jnp.clip(jnp.floor(p1), 0.0, K - 2.0)
        f0 = p0 - lo0
        f1 = p1 - lo1
        kb = lo0.astype(_I32) * K + lo1.astype(_I32)
        idx_ref[0, :] = kb * N + s
        idx_ref[1, :] = (kb + 1) * N + s
        idx_ref[2, :] = (kb + K) * N + s
        idx_ref[3, :] = (kb + K + 1) * N + s
        g0 = 1.0 - f0
        g1 = 1.0 - f1
        w_ref[0, :] = g0 * g1
        w_ref[1, :] = g0 * f1
        w_ref[2, :] = f0 * g1
        w_ref[3, :] = f0 * f1

    return pl.pallas_call(
        body,
        out_shape=[
            jax.ShapeDtypeStruct((4, E), _I32),
            jax.ShapeDtypeStruct((4, E), _F32),
        ],
    )(src, px, py)


def _mm(a, w, b, act=None, bn=400):
    n, kin = a.shape
    m = w.shape[1]
    if b is None:
        b = jnp.zeros((m,), _F32)

    def body(a_ref, w_ref, b_ref, o_ref):
        acc = jnp.dot(a_ref[:], w_ref[:], preferred_element_type=_F32,
                      precision=lax.Precision.HIGHEST)
        acc = acc + b_ref[:]
        if act == "elu":
            acc = jnp.where(acc > 0.0, acc, (jnp.exp(acc) - 1.0))
        o_ref[:] = acc

    return pl.pallas_call(
        body,
        grid=(n // bn,),
        in_specs=[
            pl.BlockSpec((bn, kin), lambda i: (i, 0)),
            pl.BlockSpec((kin, m), lambda i: (0, 0)),
            pl.BlockSpec((1, m), lambda i: (0, 0)),
        ],
        out_specs=pl.BlockSpec((bn, m), lambda i: (i, 0)),
        out_shape=jax.ShapeDtypeStruct((n, m), _F32),
    )(a, w, b.reshape(1, m))


def _mm_table(h, wpad, wacc, one_col=None, bn=400):
    n, fin = h.shape

    def body(a_ref, w_ref, o_ref):
        acc = jnp.dot(a_ref[:], w_ref[0], preferred_element_type=_F32,
                      precision=lax.Precision.HIGHEST)
        if one_col is not None:
            lane = lax.broadcasted_iota(_I32, (bn, wacc), 1)
            acc = acc + (lane == one_col).astype(_F32)
        o_ref[0] = acc

    return pl.pallas_call(
        body,
        grid=(n // bn, KK),
        in_specs=[
            pl.BlockSpec((bn, fin), lambda i, k: (i, 0)),
            pl.BlockSpec((1, fin, wacc), lambda i, k: (k, 0, 0)),
        ],
        out_specs=pl.BlockSpec((1, bn, wacc), lambda i, k: (k, i, 0)),
        out_shape=jax.ShapeDtypeStruct((KK, n, wacc), _F32),
    )(h, wpad)


def _post(p0, p1, h, root, bias, fout, rdeg=None):
    bn = 400
    first = rdeg is None

    def body(p0_ref, p1_ref, h_ref, root_ref, b_ref, *refs):
        agg = p0_ref[:] + p1_ref[:]
        if first:
            o_ref, rdeg_ref = refs
            rd = 1.0 / jnp.maximum(agg[:, fout:fout + 1], 1.0)
            rdeg_ref[:] = rd
        else:
            rd_ref, o_ref = refs[0], refs[1]
            rd = rd_ref[:]
        acc = agg[:, :fout] * rd
        acc = acc + jnp.dot(h_ref[:], root_ref[:], preferred_element_type=_F32,
                            precision=lax.Precision.HIGHEST)
        acc = acc + b_ref[:]
        o_ref[:] = jnp.where(acc > 0.0, acc, (jnp.exp(acc) - 1.0))

    wacc = p0.shape[1]
    fin = h.shape[1]
    in_specs = [
        pl.BlockSpec((bn, wacc), lambda i: (i, 0)),
        pl.BlockSpec((bn, wacc), lambda i: (i, 0)),
        pl.BlockSpec((bn, fin), lambda i: (i, 0)),
        pl.BlockSpec((fin, fout), lambda i: (0, 0)),
        pl.BlockSpec((1, fout), lambda i: (0, 0)),
    ]
    args = [p0, p1, h, root, bias.reshape(1, fout)]
    if first:
        out_specs = [
            pl.BlockSpec((bn, fout), lambda i: (i, 0)),
            pl.BlockSpec((bn, 1), lambda i: (i, 0)),
        ]
        out_shape = [
            jax.ShapeDtypeStruct((N, fout), _F32),
            jax.ShapeDtypeStruct((N, 1), _F32),
        ]
    else:
        in_specs.append(pl.BlockSpec((bn, 1), lambda i: (i, 0)))
        args.append(rdeg)
        out_specs = pl.BlockSpec((bn, fout), lambda i: (i, 0))
        out_shape = jax.ShapeDtypeStruct((N, fout), _F32)

    return pl.pallas_call(
        body,
        grid=(N // bn,),
        in_specs=in_specs,
        out_specs=out_specs,
        out_shape=out_shape,
    )(*args)


def _final_logsoftmax(a, w, b):
    n, kin = a.shape
    m = w.shape[1]
    bn = 400

    def body(a_ref, w_ref, b_ref, o_ref):
        logits = jnp.dot(a_ref[:], w_ref[:], preferred_element_type=_F32,
                         precision=lax.Precision.HIGHEST)
        logits = logits + b_ref[:]
        mx = jnp.max(logits, axis=1, keepdims=True)
        sh = logits - mx
        lse = jnp.log(jnp.sum(jnp.exp(sh), axis=1, keepdims=True))
        o_ref[:] = sh - lse

    return pl.pallas_call(
        body,
        grid=(n // bn,),
        in_specs=[
            pl.BlockSpec((bn, kin), lambda i: (i, 0)),
            pl.BlockSpec((kin, m), lambda i: (0, 0)),
            pl.BlockSpec((1, m), lambda i: (0, 0)),
        ],
        out_specs=pl.BlockSpec((bn, m), lambda i: (i, 0)),
        out_shape=jax.ShapeDtypeStruct((n, m), _F32),
    )(a, w, b.reshape(1, m))


def _sc_scatter(y, g0, g1, g2, g3, w0, w1, w2, w3, dstc, wacc, npass):
    mesh = plsc.VectorSubcoreMesh(core_axis_name="c", subcore_axis_name="s")
    nvec = wacc // 16
    nh = ((N + npass - 1) // npass + 15) // 16 * 16
    rpsh = nh // NSUB
    nacc = nh + 8
    zfull = rpsh // 16 * 16
    zrem = rpsh - zfull

    def body(y_hbm, g0_h, g1_h, g2_h, g3_h, w0_h, w1_h, w2_h, w3_h, dst_h,
             out_hbm, spill_hbm, i0, i1, i2, i3, wv0, wv1, wv2, wv3, dstv,
             r0, r1, r2, r3, msgv, zb, acc, sem):
        cid = lax.axis_index("c")
        sid = lax.axis_index("s")
        wid = sid * NCORES + cid
        base = wid * EPW
        zoff = sid * rpsh

        @pl.loop(0, 16)
        def _(r):
            for c in range(nvec):
                zb[r, pl.ds(c * 16, 16)] = jnp.zeros((16,), _F32)

        for p in range(npass):
            @pl.loop(0, zfull, step=16)
            def _(r):
                pltpu.sync_copy(zb, acc.at[pl.ds(zoff + r, 16)])

            if zrem:
                pltpu.sync_copy(zb.at[pl.ds(0, zrem)],
                                acc.at[pl.ds(zoff + zfull, zrem)])
            plsc.subcore_barrier()

            @pl.loop(0, EPW, step=EBLK)
            def _(j):
                s = base + j
                pltpu.sync_copy(dst_h.at[pl.ds(s, EBLK)], dstv)
                if p == 0:
                    pltpu.sync_copy(g0_h.at[pl.ds(s, EBLK)], i0)
                    pltpu.sync_copy(g1_h.at[pl.ds(s, EBLK)], i1)
                    pltpu.sync_copy(g2_h.at[pl.ds(s, EBLK)], i2)
                    pltpu.sync_copy(g3_h.at[pl.ds(s, EBLK)], i3)
                    pltpu.sync_copy(w0_h.at[pl.ds(s, EBLK)], wv0)
                    pltpu.sync_copy(w1_h.at[pl.ds(s, EBLK)], wv1)
                    pltpu.sync_copy(w2_h.at[pl.ds(s, EBLK)], wv2)
                    pltpu.sync_copy(w3_h.at[pl.ds(s, EBLK)], wv3)
                    d0 = pltpu.async_copy(y_hbm.at[i0], r0, sem)
                    d1 = pltpu.async_copy(y_hbm.at[i1], r1, sem)
                    d2 = pltpu.async_copy(y_hbm.at[i2], r2, sem)
                    d3 = pltpu.async_copy(y_hbm.at[i3], r3, sem)
                    d0.wait()
                    d1.wait()
                    d2.wait()
                    d3.wait()

                    @pl.loop(0, EBLK, step=16)
                    def _(e0):
                        vw0 = wv0[pl.ds(e0, 16)]
                        vw1 = wv1[pl.ds(e0, 16)]
                        vw2 = wv2[pl.ds(e0, 16)]
                        vw3 = wv3[pl.ds(e0, 16)]
                        for jj in range(16):
                            e = e0 + jj
                            a0 = vw0[jj]
                            a1 = vw1[jj]
                            a2 = vw2[jj]
                            a3 = vw3[jj]
                            for c in range(nvec):
                                sl = pl.ds(c * 16, 16)
                                msgv[e, sl] = (a0 * r0[e, sl] + a1 * r1[e, sl]
                                               + a2 * r2[e, sl]
                                               + a3 * r3[e, sl])

                    pltpu.sync_copy(msgv, spill_hbm.at[pl.ds(s, EBLK)])
                else:
                    pltpu.sync_copy(spill_hbm.at[pl.ds(s, EBLK)], msgv)

                @pl.loop(0, EBLK, step=16)
                def _(e0):
                    d = dstv[pl.ds(e0, 16)] - p * nh
                    inr = (d >= 0) & (d < nh)
                    dstv[pl.ds(e0, 16)] = jnp.where(inr, d, nh)

                pltpu.sync_copy(msgv, acc.at[dstv], add=True)

            plsc.subcore_barrier()
            pltpu.sync_copy(acc.at[pl.ds(sid * rpsh, rpsh)],
                            out_hbm.at[cid, p, sid])
            plsc.subcore_barrier()

    kfn = pl.kernel(
        body,
        out_type=[
            jax.ShapeDtypeStruct((NCORES, npass, NSUB, rpsh, wacc), _F32),
            jax.ShapeDtypeStruct((EP, wacc), _F32),
        ],
        mesh=mesh,
        scratch_types=[
            pltpu.VMEM((EBLK,), _I32),
            pltpu.VMEM((EBLK,), _I32),
            pltpu.VMEM((EBLK,), _I32),
            pltpu.VMEM((EBLK,), _I32),
            pltpu.VMEM((EBLK,), _F32),
            pltpu.VMEM((EBLK,), _F32),
            pltpu.VMEM((EBLK,), _F32),
            pltpu.VMEM((EBLK,), _F32),
            pltpu.VMEM((EBLK,), _I32),
            pltpu.VMEM((EBLK, 128), _F32),
            pltpu.VMEM((EBLK, 128), _F32),
            pltpu.VMEM((EBLK, 128), _F32),
            pltpu.VMEM((EBLK, 128), _F32),
            pltpu.VMEM((EBLK, wacc), _F32),
            pltpu.VMEM((16, wacc), _F32),
            pltpu.VMEM_SHARED((nacc, wacc), _F32),
            pltpu.SemaphoreType.DMA,
        ],
    )
    return kfn(y, g0, g1, g2, g3, w0, w1, w2, w3, dstc)


def _spline_layer(h, gparts, wparts, dstc, W, root, bias, rdeg, npass):
    fout = W.shape[2]
    first = rdeg is None
    wacc = fout + 16 if first else fout
    wpad = jnp.pad(W, ((0, 0), (0, 0), (0, 128 - fout)))
    one_col = fout if first else None
    y = _mm_table(h, wpad, 128, one_col=one_col).reshape(KK * N, 128)
    parts, _ = _sc_scatter(y, *gparts, *wparts, dstc, wacc, npass)
    nrows = parts.shape[1] * parts.shape[2] * parts.shape[3]
    parts = parts.reshape(NCORES, nrows, wacc)[:, :N, :]
    return _post(parts[0], parts[1], h, root, bias, fout, rdeg=rdeg)


def kernel(adj, x, pseudo, W0, b0, W1, root1, bias1, W2, root2, bias2,
           W3, root3, bias3, W4, b4, W5, b5):
    src = adj[0].astype(_I32)
    dstc = adj[1].astype(_I32)
    px = pseudo[:, 0].astype(_F32)
    py = pseudo[:, 1].astype(_F32)

    gidx, gw = _prep_basis(src, px, py)
    pad = EP - E
    spread = (jnp.arange(pad, dtype=_I32) * 97) % (KK * N)
    gparts = tuple(jnp.concatenate([gidx[t], spread]) for t in range(4))
    wparts = tuple(jnp.pad(gw[t], (0, pad)) for t in range(4))
    dstp = jnp.pad(dstc, (0, pad), constant_values=N)

    h = _mm(x, W0, b0, act="elu")
    h, rdeg = _spline_layer(h, gparts, wparts, dstp, W1, root1, bias1, None, 2)
    h = _spline_layer(h, gparts, wparts, dstp, W2, root2, bias2, rdeg, 2)
    h = _spline_layer(h, gparts, wparts, dstp, W3, root3, bias3, rdeg, 2)
    h = _mm(h, W4, b4, act="elu")

    mpad = 6912
    w5p = jnp.pad(W5, ((0, 0), (0, mpad - 6890)))
    b5p = jnp.concatenate([b5, jnp.full((mpad - 6890,), -1e30, _F32)])
    out = _final_logsoftmax(h, w5p, b5p)
    return out[:, :6890]
